# baseline (device time: 69216 ns/iter reference)
import jax
import jax.numpy as jnp
from jax import lax
from jax.experimental import pallas as pl
from jax.experimental.pallas import tpu as pltpu

N_DEV = 8

ROWS = (160, 176, 176)
OFFS = (0, 160, 336)
XS_OK = (True, True, False)

DIMS = ((1, 3, 4),
        (3, 4, 1),
        (4, 1, 3))

SEND_CONSTS = (
    ((0,), (0, 1), (0, 1, 2, 3)),
    ((0,), (0, 3), (0, 3, 4, 7)),
    ((0,), (0, 4), (0, 4, 1, 5)),
)
SEM_BASE = (0, 1, 3)

FP8 = jnp.float8_e4m3fn


def kernel(x, w_mat, scale_x, scale_w):
    m_per, k = x.shape
    n_per = w_mat.shape[1]
    x8 = x.astype(FP8)
    w8 = w_mat.astype(FP8)

    def body(x_ref, w_ref, sx_ref, sw_ref, out_ref,
             buf_a, buf_b, buf_c,
             send_a, recv_a, send_b, recv_b, send_c, recv_c):
        my = lax.axis_index("i")
        bufs = (buf_a, buf_b, buf_c)
        send_sems = (send_a, send_b, send_c)
        recv_sems = (recv_a, recv_b, recv_c)

        barrier = pltpu.get_barrier_semaphore()
        for mask in (1, 3, 4):
            pl.semaphore_signal(
                barrier, inc=1,
                device_id=(my ^ mask,), device_id_type=pl.DeviceIdType.MESH,
            )
        pl.semaphore_wait(barrier, 3)

        scale = sx_ref[0] * sw_ref[0]

        def make_rdma(s, phase, j):
            c = SEND_CONSTS[s][phase][j]
            slot = my ^ c
            sem_i = SEM_BASE[phase] + j
            if c == 0 and XS_OK[s]:
                src = x_ref.at[OFFS[s]:OFFS[s] + ROWS[s], :]
            else:
                src = bufs[s].at[slot]
            return pltpu.make_async_remote_copy(
                src_ref=src,
                dst_ref=bufs[s].at[slot],
                send_sem=send_sems[s].at[sem_i],
                recv_sem=recv_sems[s].at[sem_i],
                device_id=(my ^ DIMS[s][phase],),
                device_id_type=pl.DeviceIdType.MESH,
            )

        def compute(s, slot, a=None):
            if a is None:
                a = bufs[s][pl.ds(slot, 1)].reshape(ROWS[s], k)
            acc = lax.dot_general(
                a, w_ref[...],
                (((1,), (0,)), ((), ())),
                preferred_element_type=jnp.float32,
            )
            y = acc * scale
            out_ref[pl.ds(slot * m_per + OFFS[s], ROWS[s]), :] = (
                y * jax.nn.sigmoid(y))

        p1 = []
        for s in range(3):
            if not XS_OK[s]:
                bufs[s][pl.ds(my, 1)] = (
                    x_ref[OFFS[s]:OFFS[s] + ROWS[s], :].reshape(1, ROWS[s], k))
            r = make_rdma(s, 0, 0)
            r.start()
            p1.append(r)
        p2 = [[make_rdma(s, 1, 0)] for s in range(3)]
        for s in range(3):
            p2[s][0].start()

        for s in range(3):
            if XS_OK[s]:
                compute(s, my, a=x_ref[OFFS[s]:OFFS[s] + ROWS[s], :])
            else:
                compute(s, my)

        p3 = [[], [], []]
        for s in range(3):
            p1[s].wait()
            r = make_rdma(s, 1, 1)
            r.start()
            p2[s].append(r)
        for s in range(3):
            for j in (0, 1):
                r = make_rdma(s, 2, j)
                r.start()
                p3[s].append(r)
        for s in range(3):
            compute(s, my ^ DIMS[s][0])

        for s in range(3):
            p2[s][0].wait()
            p2[s][1].wait()
            for j in (2, 3):
                r = make_rdma(s, 2, j)
                r.start()
                p3[s].append(r)
        for s in range(3):
            for c in SEND_CONSTS[s][1]:
                compute(s, my ^ DIMS[s][1] ^ c)

        for j in range(4):
            for s in (2, 1, 0):
                p3[s][j].wait()
                compute(s, my ^ DIMS[s][2] ^ SEND_CONSTS[s][2][j])

    return pl.pallas_call(
        body,
        out_shape=jax.ShapeDtypeStruct((N_DEV * m_per, n_per), jnp.float32),
        in_specs=[
            pl.BlockSpec(memory_space=pltpu.VMEM),
            pl.BlockSpec(memory_space=pltpu.VMEM),
            pl.BlockSpec(memory_space=pltpu.SMEM),
            pl.BlockSpec(memory_space=pltpu.SMEM),
        ],
        out_specs=pl.BlockSpec(memory_space=pltpu.VMEM),
        scratch_shapes=[
            pltpu.VMEM((N_DEV, ROWS[0], k), FP8),
            pltpu.VMEM((N_DEV, ROWS[1], k), FP8),
            pltpu.VMEM((N_DEV, ROWS[2], k), FP8),
            pltpu.SemaphoreType.DMA((7,)),
            pltpu.SemaphoreType.DMA((7,)),
            pltpu.SemaphoreType.DMA((7,)),
            pltpu.SemaphoreType.DMA((7,)),
            pltpu.SemaphoreType.DMA((7,)),
            pltpu.SemaphoreType.DMA((7,)),
        ],
        compiler_params=pltpu.CompilerParams(collective_id=0),
    )(x8, w8, scale_x, scale_w)


# device time: 67920 ns/iter; 1.0191x vs baseline; 1.0191x over previous
import jax
import jax.numpy as jnp
from jax import lax
from jax.experimental import pallas as pl
from jax.experimental.pallas import tpu as pltpu

N_DEV = 8

ROWS = (160, 176, 176)
OFFS = (0, 160, 336)
XS_OK = (True, True, False)

DIMS = ((1, 3, 4),
        (3, 4, 1),
        (4, 1, 3))

SEND_CONSTS = (
    ((0,), (0, 1), (0, 1, 2, 3)),
    ((0,), (0, 3), (0, 3, 4, 7)),
    ((0,), (0, 4), (0, 4, 1, 5)),
)
SEM_BASE = (0, 1, 3)

FP8 = jnp.float8_e4m3fn


def kernel(x, w_mat, scale_x, scale_w):
    m_per, k = x.shape
    n_per = w_mat.shape[1]

    def body(x_ref, w_ref, sx_ref, sw_ref, out_ref,
             xs_ref, buf_a, buf_b, buf_c, w8_ref,
             send_a, recv_a, send_b, recv_b, send_c, recv_c):
        my = lax.axis_index("i")
        bufs = (buf_a, buf_b, buf_c)
        send_sems = (send_a, send_b, send_c)
        recv_sems = (recv_a, recv_b, recv_c)

        barrier = pltpu.get_barrier_semaphore()
        for mask in (1, 3, 4):
            pl.semaphore_signal(
                barrier, inc=1,
                device_id=(my ^ mask,), device_id_type=pl.DeviceIdType.MESH,
            )
        pl.semaphore_wait(barrier, 3)

        scale = sx_ref[0] * sw_ref[0]

        def make_rdma(s, phase, j):
            c = SEND_CONSTS[s][phase][j]
            slot = my ^ c
            sem_i = SEM_BASE[phase] + j
            if c == 0 and XS_OK[s]:
                src = xs_ref.at[OFFS[s]:OFFS[s] + ROWS[s], :]
            else:
                src = bufs[s].at[slot]
            return pltpu.make_async_remote_copy(
                src_ref=src,
                dst_ref=bufs[s].at[slot],
                send_sem=send_sems[s].at[sem_i],
                recv_sem=recv_sems[s].at[sem_i],
                device_id=(my ^ DIMS[s][phase],),
                device_id_type=pl.DeviceIdType.MESH,
            )

        def compute(s, slot, a=None):
            if a is None:
                a = bufs[s][pl.ds(slot, 1)].reshape(ROWS[s], k)
            acc = lax.dot_general(
                a, w8_ref[...],
                (((1,), (0,)), ((), ())),
                preferred_element_type=jnp.float32,
            )
            y = acc * scale
            out_ref[pl.ds(slot * m_per + OFFS[s], ROWS[s]), :] = (
                y * jax.nn.sigmoid(y))

        p1 = []
        for s in range(3):
            if XS_OK[s]:
                xs_ref[OFFS[s]:OFFS[s] + ROWS[s], :] = (
                    x_ref[OFFS[s]:OFFS[s] + ROWS[s], :].astype(FP8))
            else:
                bufs[s][pl.ds(my, 1)] = (
                    x_ref[OFFS[s]:OFFS[s] + ROWS[s], :]
                    .astype(FP8).reshape(1, ROWS[s], k))
            r = make_rdma(s, 0, 0)
            r.start()
            p1.append(r)
        p2 = [[make_rdma(s, 1, 0)] for s in range(3)]
        for s in range(3):
            p2[s][0].start()

        w8_ref[...] = w_ref[...].astype(FP8)
        for s in range(3):
            if XS_OK[s]:
                compute(s, my, a=xs_ref[OFFS[s]:OFFS[s] + ROWS[s], :])
            else:
                compute(s, my)

        p3 = [[], [], []]
        for s in range(3):
            p1[s].wait()
            r = make_rdma(s, 1, 1)
            r.start()
            p2[s].append(r)
        for s in range(3):
            for j in (0, 1):
                r = make_rdma(s, 2, j)
                r.start()
                p3[s].append(r)
        for s in range(3):
            compute(s, my ^ DIMS[s][0])

        for s in range(3):
            p2[s][0].wait()
            p2[s][1].wait()
            for j in (2, 3):
                r = make_rdma(s, 2, j)
                r.start()
                p3[s].append(r)
        for s in range(3):
            for c in SEND_CONSTS[s][1]:
                compute(s, my ^ DIMS[s][1] ^ c)

        for j in range(4):
            for s in (2, 1, 0):
                p3[s][j].wait()
                compute(s, my ^ DIMS[s][2] ^ SEND_CONSTS[s][2][j])

    return pl.pallas_call(
        body,
        out_shape=jax.ShapeDtypeStruct((N_DEV * m_per, n_per), jnp.float32),
        in_specs=[
            pl.BlockSpec(memory_space=pltpu.VMEM),
            pl.BlockSpec(memory_space=pltpu.VMEM),
            pl.BlockSpec(memory_space=pltpu.SMEM),
            pl.BlockSpec(memory_space=pltpu.SMEM),
        ],
        out_specs=pl.BlockSpec(memory_space=pltpu.VMEM),
        scratch_shapes=[
            pltpu.VMEM((m_per, k), FP8),
            pltpu.VMEM((N_DEV, ROWS[0], k), FP8),
            pltpu.VMEM((N_DEV, ROWS[1], k), FP8),
            pltpu.VMEM((N_DEV, ROWS[2], k), FP8),
            pltpu.VMEM((k, n_per), FP8),
            pltpu.SemaphoreType.DMA((7,)),
            pltpu.SemaphoreType.DMA((7,)),
            pltpu.SemaphoreType.DMA((7,)),
            pltpu.SemaphoreType.DMA((7,)),
            pltpu.SemaphoreType.DMA((7,)),
            pltpu.SemaphoreType.DMA((7,)),
        ],
        compiler_params=pltpu.CompilerParams(collective_id=0),
    )(x, w_mat, scale_x, scale_w)
